# Initial kernel scaffold; baseline (speedup 1.0000x reference)
import jax
import jax.numpy as jnp
from jax.experimental import pallas as pl


def kernel(node_features, edge_features, senders, receivers, eW1, eb1, eW2, eb2, eg, ebeta, nW1, nb1, nW2, nb2, ng, nbeta):
    raise NotImplementedError("write your pallas kernel here")


# speedup vs baseline: 3.5203x; 3.5203x over previous
import functools

import jax
import jax.numpy as jnp
from jax import lax
from jax.experimental import pallas as pl
from jax.experimental.pallas import tpu as pltpu
from jax.experimental.pallas import tpu_sc as plsc

N = 10000
E = 320000
D = 128

NC = 2
NS = 16
NW = NC * NS
EW = E // NW
GB = 80
NG = EW // GB

F32 = jnp.float32


def _ln_rows(x, g, b):
    m = jnp.mean(x, axis=-1, keepdims=True)
    v = jnp.mean((x - m) ** 2, axis=-1, keepdims=True)
    return (x - m) / jnp.sqrt(v + 1e-5) * g + b


def _proj_body(node_ref, w1_ref, ps_ref, pd_ref):
    x = node_ref[...]
    ps_ref[...] = jnp.dot(x, w1_ref[0:D, :], preferred_element_type=F32)
    pd_ref[...] = jnp.dot(x, w1_ref[D:2 * D, :], preferred_element_type=F32)


def _project(node, eW1):
    blk = 2000
    return pl.pallas_call(
        _proj_body,
        grid=(N // blk,),
        in_specs=[
            pl.BlockSpec((blk, D), lambda i: (i, 0)),
            pl.BlockSpec((3 * D, D), lambda i: (0, 0)),
        ],
        out_specs=[
            pl.BlockSpec((blk, D), lambda i: (i, 0)),
            pl.BlockSpec((blk, D), lambda i: (i, 0)),
        ],
        out_shape=[
            jax.ShapeDtypeStruct((N, D), F32),
            jax.ShapeDtypeStruct((N, D), F32),
        ],
    )(node, eW1)


def _gather_body(ps_hbm, pd_hbm, snd_hbm, rcv_hbm, g_hbm,
                 idx_s, idx_r, rows_a, rows_b, sem_a, sem_b):
    wid = lax.axis_index("s") * NC + lax.axis_index("c")
    ebase = wid * EW
    pltpu.sync_copy(snd_hbm.at[wid], idx_s)
    pltpu.sync_copy(rcv_hbm.at[wid], idx_r)

    def group(g, carry):
        cp_a = pltpu.async_copy(ps_hbm.at[idx_s.at[g]], rows_a, sem_a)
        cp_b = pltpu.async_copy(pd_hbm.at[idx_r.at[g]], rows_b, sem_b)
        cp_a.wait()
        cp_b.wait()

        def addrow(i, c):
            for k in range(D // 16):
                sl = pl.ds(k * 16, 16)
                rows_a[i, sl] = rows_a[i, sl] + rows_b[i, sl]
            return c

        lax.fori_loop(0, GB, addrow, 0)
        pltpu.sync_copy(rows_a, g_hbm.at[pl.ds(ebase + g * GB, GB)])
        return carry

    lax.fori_loop(0, NG, group, 0)


def _gather(ps, pd, snd2d, rcv2d):
    mesh = plsc.VectorSubcoreMesh(core_axis_name="c", subcore_axis_name="s")
    f = functools.partial(
        pl.kernel,
        out_type=jax.ShapeDtypeStruct((E, D), F32),
        mesh=mesh,
        scratch_types=[
            pltpu.VMEM((NG, GB), jnp.int32),
            pltpu.VMEM((NG, GB), jnp.int32),
            pltpu.VMEM((GB, D), F32),
            pltpu.VMEM((GB, D), F32),
            pltpu.SemaphoreType.DMA,
            pltpu.SemaphoreType.DMA,
        ],
    )(_gather_body)
    return f(ps, pd, snd2d, rcv2d)


def _edge_body(e_ref, g_ref, w1_ref, b1_ref, w2_ref, b2_ref, gm_ref, bt_ref,
               h_ref, out_ref):
    x = e_ref[...]
    h1 = jnp.dot(x, w1_ref[2 * D:3 * D, :], preferred_element_type=F32)
    h1 = jnp.maximum(h1 + g_ref[...] + b1_ref[...], 0.0)
    h2 = jnp.dot(h1, w2_ref[...], preferred_element_type=F32) + b2_ref[...]
    h = _ln_rows(h2, gm_ref[...], bt_ref[...])
    h_ref[...] = h
    out_ref[...] = h + x


def _edge_mlp(edge, g, eW1, eb1, eW2, eb2, eg, ebeta):
    blk = 2000
    vec = lambda i: (0, 0)
    return pl.pallas_call(
        _edge_body,
        grid=(E // blk,),
        in_specs=[
            pl.BlockSpec((blk, D), lambda i: (i, 0)),
            pl.BlockSpec((blk, D), lambda i: (i, 0)),
            pl.BlockSpec((3 * D, D), vec),
            pl.BlockSpec((1, D), vec),
            pl.BlockSpec((D, D), vec),
            pl.BlockSpec((1, D), vec),
            pl.BlockSpec((1, D), vec),
            pl.BlockSpec((1, D), vec),
        ],
        out_specs=[
            pl.BlockSpec((blk, D), lambda i: (i, 0)),
            pl.BlockSpec((blk, D), lambda i: (i, 0)),
        ],
        out_shape=[
            jax.ShapeDtypeStruct((E, D), F32),
            jax.ShapeDtypeStruct((E, D), F32),
        ],
    )(edge, g, eW1, eb1.reshape(1, D), eW2, eb2.reshape(1, D),
      eg.reshape(1, D), ebeta.reshape(1, D))


def _scatter_body(h_hbm, rcv_hbm, zero_hbm, agg_hbm,
                  idx_r, rows, agg_sh, sem):
    c = lax.axis_index("c")
    s = lax.axis_index("s")
    wid = s * NC + c
    ebase = wid * EW

    @pl.when(s == 0)
    def _():
        pltpu.sync_copy(zero_hbm, agg_sh)

    plsc.subcore_barrier()
    pltpu.sync_copy(rcv_hbm.at[wid], idx_r)

    def group(g, carry):
        cp = pltpu.async_copy(h_hbm.at[pl.ds(ebase + g * GB, GB)], rows, sem)
        cp.wait()
        pltpu.sync_copy(rows, agg_sh.at[idx_r.at[g]], add=True)
        return carry

    lax.fori_loop(0, NG, group, 0)
    plsc.subcore_barrier()
    @pl.when(s < NS - 1)
    def _():
        pltpu.sync_copy(agg_sh.at[pl.ds(s * 624, 624)],
                        agg_hbm.at[c].at[pl.ds(s * 624, 624)])

    @pl.when(s == NS - 1)
    def _():
        pltpu.sync_copy(agg_sh.at[pl.ds(15 * 624, N - 15 * 624)],
                        agg_hbm.at[c].at[pl.ds(15 * 624, N - 15 * 624)])


def _scatter(h, rcv2d, zeros):
    mesh = plsc.VectorSubcoreMesh(core_axis_name="c", subcore_axis_name="s")
    f = functools.partial(
        pl.kernel,
        out_type=jax.ShapeDtypeStruct((NC, N, D), F32),
        mesh=mesh,
        scratch_types=[
            pltpu.VMEM((NG, GB), jnp.int32),
            pltpu.VMEM((GB, D), F32),
            pltpu.VMEM_SHARED((N, D), F32),
            pltpu.SemaphoreType.DMA,
        ],
    )(_scatter_body)
    return f(h, rcv2d, zeros)


def _node_body(n_ref, aa_ref, ab_ref, w1_ref, b1_ref, w2_ref, b2_ref,
               gm_ref, bt_ref, out_ref):
    x = n_ref[...]
    agg = aa_ref[0] + ab_ref[0]
    h1 = (jnp.dot(x, w1_ref[0:D, :], preferred_element_type=F32)
          + jnp.dot(agg, w1_ref[D:2 * D, :], preferred_element_type=F32))
    h1 = jnp.maximum(h1 + b1_ref[...], 0.0)
    h2 = jnp.dot(h1, w2_ref[...], preferred_element_type=F32) + b2_ref[...]
    out_ref[...] = _ln_rows(h2, gm_ref[...], bt_ref[...]) + x


def _node_mlp(node, aggs, nW1, nb1, nW2, nb2, ng, nbeta):
    blk = 2000
    vec = lambda i: (0, 0)
    return pl.pallas_call(
        _node_body,
        grid=(N // blk,),
        in_specs=[
            pl.BlockSpec((blk, D), lambda i: (i, 0)),
            pl.BlockSpec((1, blk, D), lambda i: (0, i, 0)),
            pl.BlockSpec((1, blk, D), lambda i: (1, i, 0)),
            pl.BlockSpec((2 * D, D), vec),
            pl.BlockSpec((1, D), vec),
            pl.BlockSpec((D, D), vec),
            pl.BlockSpec((1, D), vec),
            pl.BlockSpec((1, D), vec),
            pl.BlockSpec((1, D), vec),
        ],
        out_specs=pl.BlockSpec((blk, D), lambda i: (i, 0)),
        out_shape=jax.ShapeDtypeStruct((N, D), F32),
    )(node, aggs, aggs, nW1, nb1.reshape(1, D), nW2, nb2.reshape(1, D),
      ng.reshape(1, D), nbeta.reshape(1, D))


def kernel(node_features, edge_features, senders, receivers,
           eW1, eb1, eW2, eb2, eg, ebeta,
           nW1, nb1, nW2, nb2, ng, nbeta):
    snd2d = senders.reshape(NW, NG, GB)
    rcv2d = receivers.reshape(NW, NG, GB)
    ps, pd = _project(node_features, eW1)
    g = _gather(ps, pd, snd2d, rcv2d)
    h, new_edge = _edge_mlp(edge_features, g, eW1, eb1, eW2, eb2, eg, ebeta)
    zeros = jnp.zeros((N, D), F32)
    aggs = _scatter(h, rcv2d, zeros)
    new_node = _node_mlp(node_features, aggs, nW1, nb1, nW2, nb2, ng, nbeta)
    return (new_node, new_edge)


# speedup vs baseline: 4.4149x; 1.2541x over previous
import functools

import jax
import jax.numpy as jnp
from jax import lax
from jax.experimental import pallas as pl
from jax.experimental.pallas import tpu as pltpu
from jax.experimental.pallas import tpu_sc as plsc

N = 10000
E = 320000
D = 128

NC = 2
NS = 16
NW = NC * NS
EW = E // NW
GB = 80
NG = EW // GB

F32 = jnp.float32


def _ln_rows(x, g, b):
    m = jnp.mean(x, axis=-1, keepdims=True)
    v = jnp.mean((x - m) ** 2, axis=-1, keepdims=True)
    return (x - m) / jnp.sqrt(v + 1e-5) * g + b


def _proj_body(node_ref, w1_ref, ps_ref, pd_ref):
    x = node_ref[...]
    ps_ref[...] = jnp.dot(x, w1_ref[0:D, :], preferred_element_type=F32)
    pd_ref[...] = jnp.dot(x, w1_ref[D:2 * D, :], preferred_element_type=F32)


def _project(node, eW1):
    blk = 2000
    return pl.pallas_call(
        _proj_body,
        grid=(N // blk,),
        in_specs=[
            pl.BlockSpec((blk, D), lambda i: (i, 0)),
            pl.BlockSpec((3 * D, D), lambda i: (0, 0)),
        ],
        out_specs=[
            pl.BlockSpec((blk, D), lambda i: (i, 0)),
            pl.BlockSpec((blk, D), lambda i: (i, 0)),
        ],
        out_shape=[
            jax.ShapeDtypeStruct((N, D), F32),
            jax.ShapeDtypeStruct((N, D), F32),
        ],
    )(node, eW1)


def _gather_body(ps_hbm, pd_hbm, snd_hbm, rcv_hbm, g_hbm,
                 idx_s, idx_r, rows_a, rows_b, sem_a, sem_b):
    wid = lax.axis_index("s") * NC + lax.axis_index("c")
    ebase = wid * EW
    pltpu.sync_copy(snd_hbm.at[wid], idx_s)
    pltpu.sync_copy(rcv_hbm.at[wid], idx_r)

    def issue(g, buf):
        cp_a = pltpu.async_copy(ps_hbm.at[idx_s.at[g]], rows_a.at[buf], sem_a.at[buf])
        cp_b = pltpu.async_copy(pd_hbm.at[idx_r.at[g]], rows_b.at[buf], sem_b.at[buf])
        return cp_a, cp_b

    def drain(g, buf):
        cp_a, cp_b = (
            pltpu.make_async_copy(ps_hbm.at[idx_s.at[g]], rows_a.at[buf], sem_a.at[buf]),
            pltpu.make_async_copy(pd_hbm.at[idx_r.at[g]], rows_b.at[buf], sem_b.at[buf]),
        )
        cp_a.wait()
        cp_b.wait()

    def addstore(g, buf):
        def addrow(i, c):
            for k in range(D // 16):
                sl = pl.ds(k * 16, 16)
                rows_a[buf, i, sl] = rows_a[buf, i, sl] + rows_b[buf, i, sl]
            return c

        lax.fori_loop(0, GB, addrow, 0)
        pltpu.sync_copy(rows_a.at[buf], g_hbm.at[pl.ds(ebase + g * GB, GB)])

    issue(0, 0)

    def pair(t, carry):
        g0 = 2 * t
        drain(g0, 0)
        issue(g0 + 1, 1)
        addstore(g0, 0)
        drain(g0 + 1, 1)
        issue(g0 + 2, 0)
        addstore(g0 + 1, 1)
        return carry

    lax.fori_loop(0, (NG - 1) // 2, pair, 0)
    drain(NG - 1, 0)
    addstore(NG - 1, 0)


def _gather(ps, pd, snd2d, rcv2d):
    mesh = plsc.VectorSubcoreMesh(core_axis_name="c", subcore_axis_name="s")
    f = functools.partial(
        pl.kernel,
        out_type=jax.ShapeDtypeStruct((E, D), F32),
        mesh=mesh,
        scratch_types=[
            pltpu.VMEM((NG, GB), jnp.int32),
            pltpu.VMEM((NG, GB), jnp.int32),
            pltpu.VMEM((2, GB, D), F32),
            pltpu.VMEM((2, GB, D), F32),
            pltpu.SemaphoreType.DMA((2,)),
            pltpu.SemaphoreType.DMA((2,)),
        ],
    )(_gather_body)
    return f(ps, pd, snd2d, rcv2d)


def _edge_body(e_ref, g_ref, w1_ref, b1_ref, w2_ref, b2_ref, gm_ref, bt_ref,
               h_ref, out_ref):
    x = e_ref[...]
    h1 = jnp.dot(x, w1_ref[2 * D:3 * D, :], preferred_element_type=F32)
    h1 = jnp.maximum(h1 + g_ref[...] + b1_ref[...], 0.0)
    h2 = jnp.dot(h1, w2_ref[...], preferred_element_type=F32) + b2_ref[...]
    h = _ln_rows(h2, gm_ref[...], bt_ref[...])
    h_ref[...] = h
    out_ref[...] = h + x


def _edge_mlp(edge, g, eW1, eb1, eW2, eb2, eg, ebeta):
    blk = 2000
    vec = lambda i: (0, 0)
    return pl.pallas_call(
        _edge_body,
        grid=(E // blk,),
        in_specs=[
            pl.BlockSpec((blk, D), lambda i: (i, 0)),
            pl.BlockSpec((blk, D), lambda i: (i, 0)),
            pl.BlockSpec((3 * D, D), vec),
            pl.BlockSpec((1, D), vec),
            pl.BlockSpec((D, D), vec),
            pl.BlockSpec((1, D), vec),
            pl.BlockSpec((1, D), vec),
            pl.BlockSpec((1, D), vec),
        ],
        out_specs=[
            pl.BlockSpec((blk, D), lambda i: (i, 0)),
            pl.BlockSpec((blk, D), lambda i: (i, 0)),
        ],
        out_shape=[
            jax.ShapeDtypeStruct((E, D), F32),
            jax.ShapeDtypeStruct((E, D), F32),
        ],
    )(edge, g, eW1, eb1.reshape(1, D), eW2, eb2.reshape(1, D),
      eg.reshape(1, D), ebeta.reshape(1, D))


def _scatter_body(h_hbm, rcv_hbm, zero_hbm, agg_hbm,
                  idx_r, rows, agg_sh, sem):
    c = lax.axis_index("c")
    s = lax.axis_index("s")
    wid = s * NC + c
    ebase = wid * EW

    @pl.when(s == 0)
    def _():
        pltpu.sync_copy(zero_hbm, agg_sh)

    plsc.subcore_barrier()
    pltpu.sync_copy(rcv_hbm.at[wid], idx_r)

    def issue(g, buf):
        pltpu.async_copy(h_hbm.at[pl.ds(ebase + g * GB, GB)], rows.at[buf],
                         sem.at[buf])

    def drain(g, buf):
        pltpu.make_async_copy(h_hbm.at[pl.ds(ebase + g * GB, GB)], rows.at[buf],
                              sem.at[buf]).wait()

    def scat(g, buf):
        pltpu.sync_copy(rows.at[buf], agg_sh.at[idx_r.at[g]], add=True)

    issue(0, 0)

    def pair(t, carry):
        g0 = 2 * t
        drain(g0, 0)
        issue(g0 + 1, 1)
        scat(g0, 0)
        drain(g0 + 1, 1)
        issue(g0 + 2, 0)
        scat(g0 + 1, 1)
        return carry

    lax.fori_loop(0, (NG - 1) // 2, pair, 0)
    drain(NG - 1, 0)
    scat(NG - 1, 0)
    plsc.subcore_barrier()
    @pl.when(s < NS - 1)
    def _():
        pltpu.sync_copy(agg_sh.at[pl.ds(s * 624, 624)],
                        agg_hbm.at[c].at[pl.ds(s * 624, 624)])

    @pl.when(s == NS - 1)
    def _():
        pltpu.sync_copy(agg_sh.at[pl.ds(15 * 624, N - 15 * 624)],
                        agg_hbm.at[c].at[pl.ds(15 * 624, N - 15 * 624)])


def _scatter(h, rcv2d, zeros):
    mesh = plsc.VectorSubcoreMesh(core_axis_name="c", subcore_axis_name="s")
    f = functools.partial(
        pl.kernel,
        out_type=jax.ShapeDtypeStruct((NC, N, D), F32),
        mesh=mesh,
        scratch_types=[
            pltpu.VMEM((NG, GB), jnp.int32),
            pltpu.VMEM((2, GB, D), F32),
            pltpu.VMEM_SHARED((N, D), F32),
            pltpu.SemaphoreType.DMA((2,)),
        ],
    )(_scatter_body)
    return f(h, rcv2d, zeros)


def _node_body(n_ref, aa_ref, ab_ref, w1_ref, b1_ref, w2_ref, b2_ref,
               gm_ref, bt_ref, out_ref):
    x = n_ref[...]
    agg = aa_ref[0] + ab_ref[0]
    h1 = (jnp.dot(x, w1_ref[0:D, :], preferred_element_type=F32)
          + jnp.dot(agg, w1_ref[D:2 * D, :], preferred_element_type=F32))
    h1 = jnp.maximum(h1 + b1_ref[...], 0.0)
    h2 = jnp.dot(h1, w2_ref[...], preferred_element_type=F32) + b2_ref[...]
    out_ref[...] = _ln_rows(h2, gm_ref[...], bt_ref[...]) + x


def _node_mlp(node, aggs, nW1, nb1, nW2, nb2, ng, nbeta):
    blk = 2000
    vec = lambda i: (0, 0)
    return pl.pallas_call(
        _node_body,
        grid=(N // blk,),
        in_specs=[
            pl.BlockSpec((blk, D), lambda i: (i, 0)),
            pl.BlockSpec((1, blk, D), lambda i: (0, i, 0)),
            pl.BlockSpec((1, blk, D), lambda i: (1, i, 0)),
            pl.BlockSpec((2 * D, D), vec),
            pl.BlockSpec((1, D), vec),
            pl.BlockSpec((D, D), vec),
            pl.BlockSpec((1, D), vec),
            pl.BlockSpec((1, D), vec),
            pl.BlockSpec((1, D), vec),
        ],
        out_specs=pl.BlockSpec((blk, D), lambda i: (i, 0)),
        out_shape=jax.ShapeDtypeStruct((N, D), F32),
    )(node, aggs, aggs, nW1, nb1.reshape(1, D), nW2, nb2.reshape(1, D),
      ng.reshape(1, D), nbeta.reshape(1, D))


def kernel(node_features, edge_features, senders, receivers,
           eW1, eb1, eW2, eb2, eg, ebeta,
           nW1, nb1, nW2, nb2, ng, nbeta):
    snd2d = senders.reshape(NW, NG, GB)
    rcv2d = receivers.reshape(NW, NG, GB)
    ps, pd = _project(node_features, eW1)
    g = _gather(ps, pd, snd2d, rcv2d)
    h, new_edge = _edge_mlp(edge_features, g, eW1, eb1, eW2, eb2, eg, ebeta)
    zeros = jnp.zeros((N, D), F32)
    aggs = _scatter(h, rcv2d, zeros)
    new_node = _node_mlp(node_features, aggs, nW1, nb1, nW2, nb2, ng, nbeta)
    return (new_node, new_edge)
